# baseline (speedup 1.0000x reference)
import functools

import jax
import jax.numpy as jnp
from jax import lax
from jax.experimental import pallas as pl
from jax.experimental.pallas import tpu as pltpu
from jax.experimental.pallas import tpu_sc as plsc

_NC = 2
_NS = 16
_W = 128
_CW = 64
_BLK = 40
_NB = 4
_K0 = 8
_K1 = 0


def _deg_body(dst_hbm, ew_hbm, out_hbm, idx_v, ew_v, stage_v, acc_sh):
    c = lax.axis_index("c")
    s = lax.axis_index("s")
    n = stage_v.shape[0]
    rows = idx_v.shape[0]

    @pl.when(s == 0)
    def _zero():
        def zb(i, carry):
            stage_v[pl.ds(i * 16, 16)] = jnp.zeros((16,), jnp.float32)
            return carry
        lax.fori_loop(0, n // 16, zb, 0)
        pltpu.sync_copy(stage_v, acc_sh)

    plsc.subcore_barrier()

    base = (c * _NS + s) * rows
    pltpu.sync_copy(dst_hbm.at[pl.ds(base, rows)], idx_v)
    pltpu.sync_copy(ew_hbm.at[pl.ds(base, rows)], ew_v)

    def body(j, carry):
        pltpu.sync_copy(ew_v.at[j], acc_sh.at[idx_v.at[j]], add=True)
        return carry
    lax.fori_loop(0, rows, body, 0)

    plsc.subcore_barrier()

    @pl.when(s == 0)
    def _writeback():
        pltpu.sync_copy(acc_sh, stage_v)
        pltpu.sync_copy(stage_v, out_hbm.at[pl.ds(c * n, n)])


def _deg_call(dst2d, ew2d, n2):
    rows = dst2d.shape[0] // (_NC * _NS)
    mesh = plsc.VectorSubcoreMesh(core_axis_name="c", subcore_axis_name="s")
    kfn = pl.kernel(
        functools.partial(_deg_body),
        mesh=mesh,
        out_type=jax.ShapeDtypeStruct((_NC * n2,), jnp.float32),
        scratch_types=[
            pltpu.VMEM((rows, _W), jnp.int32),
            pltpu.VMEM((rows, _W), jnp.float32),
            pltpu.VMEM((n2,), jnp.float32),
            pltpu.VMEM_SHARED((n2,), jnp.float32),
        ],
    )
    return kfn(dst2d, ew2d)


def _scale(buf, ewv, r):
    def grp(eb, c2):
        wv = ewv[r, pl.ds(eb * 16, 16)]
        for l in range(16):
            wb = lax.broadcast_in_dim(wv[l], (16,), ())
            e = eb * 16 + l
            for q in range(8):
                buf[e, pl.ds(q * 16, 16)] = buf[e, pl.ds(q * 16, 16)] * wb
        return c2
    lax.fori_loop(0, buf.shape[0] // 16, grp, 0)


def _msg_body(u_hbm, src_hbm, dst_hbm, ew_hbm, out_hbm,
              srcv, dstv, ewv, buf0, buf1, buf2, buf3,
              gs0, gs1, gs2, gs3, ss0, ss1, ss2, ss3,
              acc_sh):
    c = lax.axis_index("c")
    s = lax.axis_index("s")
    bufs = (buf0, buf1, buf2, buf3)
    gs = (gs0, gs1, gs2, gs3)
    ss = (ss0, ss1, ss2, ss3)
    nblk = jnp.where(c == 0, _K0, _K1)
    stripe = acc_sh.shape[0] // _NS
    sizes = [_CW] * (stripe // _CW) + ([stripe % _CW] if stripe % _CW else [])

    def zb(i, carry):
        for q in range(8):
            buf0[i, pl.ds(q * 16, 16)] = jnp.zeros((16,), jnp.float32)
        return carry
    lax.fori_loop(0, _CW, zb, 0)
    off = 0
    for sz in sizes:
        pltpu.sync_copy(buf0.at[pl.ds(0, sz)],
                        acc_sh.at[pl.ds(s * stripe + off, sz)])
        off += sz
    plsc.subcore_barrier()

    base = jnp.where(c == 0, _K0 * s, _K0 * _NS + _K1 * s) * _BLK
    dummy = u_hbm.at[pl.ds(0, _CW)]

    def wait_d(sem, buf):
        pltpu.make_async_copy(dummy, buf, sem).wait()

    def block(b, carry):
        @pl.when(b > 0)
        def _drain():
            for k in range(_NB):
                wait_d(ss[k], bufs[k])
        pltpu.sync_copy(src_hbm.at[pl.ds(base + b * _BLK, _BLK)], srcv)
        pltpu.sync_copy(dst_hbm.at[pl.ds(base + b * _BLK, _BLK)], dstv)
        pltpu.sync_copy(ew_hbm.at[pl.ds(base + b * _BLK, _BLK)], ewv)
        for k in range(_NB):
            pltpu.async_copy(u_hbm.at[srcv.at[k]], bufs[k], gs[k])

        def quad(q, c2):
            for k in range(_NB):
                r = _NB * q + k
                wait_d(gs[k], bufs[k])
                _scale(bufs[k], ewv, r)
                pltpu.async_copy(bufs[k], acc_sh.at[dstv.at[r]], ss[k],
                                 add=True)

                @pl.when(q < _BLK // _NB - 1)
                def _next():
                    wait_d(ss[k], bufs[k])
                    pltpu.async_copy(u_hbm.at[srcv.at[r + _NB]], bufs[k],
                                     gs[k])
            return c2
        lax.fori_loop(0, _BLK // _NB, quad, 0)
        return carry
    lax.fori_loop(0, nblk, block, 0)

    @pl.when(nblk > 0)
    def _final_drain():
        for k in range(_NB):
            wait_d(ss[k], bufs[k])
    plsc.subcore_barrier()

    off = 0
    for sz in sizes:
        pltpu.sync_copy(acc_sh.at[pl.ds(s * stripe + off, sz)],
                        buf0.at[pl.ds(0, sz)])
        pltpu.sync_copy(buf0.at[pl.ds(0, sz)],
                        out_hbm.at[c, pl.ds(s * stripe + off, sz)])
        off += sz


def _msg_call(u, src2d, dst2d, ew2d, n2):
    n, d = u.shape
    mesh = plsc.VectorSubcoreMesh(core_axis_name="c", subcore_axis_name="s")
    kfn = pl.kernel(
        functools.partial(_msg_body),
        mesh=mesh,
        out_type=jax.ShapeDtypeStruct((_NC, n2, d), jnp.float32),
        scratch_types=[
            pltpu.VMEM((_BLK, _CW), jnp.int32),
            pltpu.VMEM((_BLK, _CW), jnp.int32),
            pltpu.VMEM((_BLK, _CW), jnp.float32),
            pltpu.VMEM((_CW, d), jnp.float32),
            pltpu.VMEM((_CW, d), jnp.float32),
            pltpu.VMEM((_CW, d), jnp.float32),
            pltpu.VMEM((_CW, d), jnp.float32),
            pltpu.SemaphoreType.DMA,
            pltpu.SemaphoreType.DMA,
            pltpu.SemaphoreType.DMA,
            pltpu.SemaphoreType.DMA,
            pltpu.SemaphoreType.DMA,
            pltpu.SemaphoreType.DMA,
            pltpu.SemaphoreType.DMA,
            pltpu.SemaphoreType.DMA,
            pltpu.VMEM_SHARED((n2, d), jnp.float32),
        ],
    )
    return kfn(u, src2d, dst2d, ew2d)


def _k2_body(degp_ref, x_ref, w_ref, u_ref, dinv_ref):
    deg = degp_ref[0] + degp_ref[1] + 1.0
    dinv = jnp.where(deg > 0, lax.rsqrt(deg), 0.0)
    dinv_ref[...] = dinv
    u_ref[...] = jnp.dot(x_ref[...], w_ref[...],
                         preferred_element_type=jnp.float32) * dinv[:, None]


def _k4_body(mp_ref, u_ref, dinv_ref, b_ref, w_ref, u2_ref):
    dinv = dinv_ref[...]
    h = (mp_ref[0] + mp_ref[1] + u_ref[...]) * dinv[:, None] + b_ref[...][None, :]
    h = jnp.maximum(h, 0.0)
    u2_ref[...] = jnp.dot(h, w_ref[...],
                          preferred_element_type=jnp.float32) * dinv[:, None]


def _k5_body(mp_ref, u_ref, dinv_ref, b_ref, batch_ref, md_ref,
             wm_ref, bm_ref, wfp_ref, wfm_ref, bf_ref, o_ref):
    n = u_ref.shape[0]
    g = md_ref.shape[0]
    dinv = dinv_ref[...]
    h = (mp_ref[0] + mp_ref[1] + u_ref[...]) * dinv[:, None] + b_ref[...][None, :]
    h = jnp.maximum(h, 0.0)
    bt = batch_ref[...]
    gi = lax.broadcasted_iota(jnp.int32, (g, n), 0)
    onehot = (gi == bt[None, :]).astype(jnp.float32)
    sums = jnp.dot(onehot, h, preferred_element_type=jnp.float32)
    counts = jnp.sum(onehot, axis=1)
    pooled = sums / jnp.maximum(counts, 1.0)[:, None]
    first_node = jnp.sum((bt[None, :] < gi).astype(jnp.int32), axis=1)
    idx = lax.rem(first_node, g)
    oh2 = (lax.broadcasted_iota(jnp.int32, (g, g), 1) == idx[:, None]
           ).astype(jnp.float32)
    md = jnp.dot(oh2, md_ref[...], preferred_element_type=jnp.float32)
    md = jnp.maximum(jnp.dot(md, wm_ref[...],
                             preferred_element_type=jnp.float32)
                     + bm_ref[...][None, :], 0.0)
    o_ref[...] = (jnp.dot(pooled, wfp_ref[...], preferred_element_type=jnp.float32)
                  + jnp.dot(md, wfm_ref[...], preferred_element_type=jnp.float32)
                  + bf_ref[...][None, :])


def kernel(x, edge_index, edge_attr, batch, metadata, W1, b1, W2, b2,
           Wm, bm, Wf, bf):
    n, d = x.shape
    e = edge_attr.shape[0]
    g = metadata.shape[0]
    unit = _NS * (_K0 + _K1) * _BLK * _CW
    ep = -(-e // unit) * unit
    pad = ep - e
    src_p = jnp.concatenate([edge_index[0],
                             jnp.zeros((pad,), edge_index.dtype)])
    dst_p = jnp.concatenate([edge_index[1],
                             jnp.zeros((pad,), edge_index.dtype)])
    ew_p = jnp.concatenate([edge_attr, jnp.zeros((pad,), edge_attr.dtype)])
    src2d = src_p.reshape(-1, _CW)
    dst2d = dst_p.reshape(-1, _CW)
    ew2d = ew_p.reshape(-1, _CW)

    n2 = ((n + _W - 1) // _W) * _W
    degp = _deg_call(dst_p.reshape(-1, _W), ew_p.reshape(-1, _W),
                     n2).reshape(_NC, n2)[:, :n]

    u1, dinv = pl.pallas_call(
        _k2_body,
        out_shape=[jax.ShapeDtypeStruct((n, d), jnp.float32),
                   jax.ShapeDtypeStruct((n,), jnp.float32)],
    )(degp, x, W1)

    m1 = _msg_call(u1, src2d, dst2d, ew2d, n2)[:, :n]

    u2 = pl.pallas_call(
        _k4_body,
        out_shape=jax.ShapeDtypeStruct((n, d), jnp.float32),
    )(m1, u1, dinv, b1, W2)

    m2 = _msg_call(u2, src2d, dst2d, ew2d, n2)[:, :n]

    out = pl.pallas_call(
        _k5_body,
        out_shape=jax.ShapeDtypeStruct((g, Wf.shape[1]), jnp.float32),
    )(m2, u2, dinv, b2, batch, metadata, Wm, bm, Wf[:d], Wf[d:], bf)
    return out


# speedup vs baseline: 1.6097x; 1.6097x over previous
import functools

import jax
import jax.numpy as jnp
from jax import lax
from jax.experimental import pallas as pl
from jax.experimental.pallas import tpu as pltpu
from jax.experimental.pallas import tpu_sc as plsc

_NC = 2
_NS = 16
_W = 128
_CW = 128
_BLK = 16
_NB = 2
_K0 = 9
_K1 = 1


def _deg_body(dst_hbm, ew_hbm, out_hbm, idx_v, ew_v, stage_v, acc_sh):
    c = lax.axis_index("c")
    s = lax.axis_index("s")
    n = stage_v.shape[0]
    rows = idx_v.shape[0]

    @pl.when(s == 0)
    def _zero():
        def zb(i, carry):
            stage_v[pl.ds(i * 16, 16)] = jnp.zeros((16,), jnp.float32)
            return carry
        lax.fori_loop(0, n // 16, zb, 0)
        pltpu.sync_copy(stage_v, acc_sh)

    plsc.subcore_barrier()

    base = (c * _NS + s) * rows
    pltpu.sync_copy(dst_hbm.at[pl.ds(base, rows)], idx_v)
    pltpu.sync_copy(ew_hbm.at[pl.ds(base, rows)], ew_v)

    def body(j, carry):
        pltpu.sync_copy(ew_v.at[j], acc_sh.at[idx_v.at[j]], add=True)
        return carry
    lax.fori_loop(0, rows, body, 0)

    plsc.subcore_barrier()

    @pl.when(s == 0)
    def _writeback():
        pltpu.sync_copy(acc_sh, stage_v)
        pltpu.sync_copy(stage_v, out_hbm.at[pl.ds(c * n, n)])


def _deg_call(dst2d, ew2d, n2):
    rows = dst2d.shape[0] // (_NC * _NS)
    mesh = plsc.VectorSubcoreMesh(core_axis_name="c", subcore_axis_name="s")
    kfn = pl.kernel(
        functools.partial(_deg_body),
        mesh=mesh,
        out_type=jax.ShapeDtypeStruct((_NC * n2,), jnp.float32),
        scratch_types=[
            pltpu.VMEM((rows, _W), jnp.int32),
            pltpu.VMEM((rows, _W), jnp.float32),
            pltpu.VMEM((n2,), jnp.float32),
            pltpu.VMEM_SHARED((n2,), jnp.float32),
        ],
    )
    return kfn(dst2d, ew2d)


def _scale(buf, ewv, r):
    def grp(eb, c2):
        wv = ewv[r, pl.ds(eb * 16, 16)]
        for l in range(16):
            wb = lax.broadcast_in_dim(wv[l], (16,), ())
            e = eb * 16 + l
            for q in range(8):
                buf[e, pl.ds(q * 16, 16)] = buf[e, pl.ds(q * 16, 16)] * wb
        return c2
    lax.fori_loop(0, buf.shape[0] // 16, grp, 0)


def _msg_body(u_hbm, src_hbm, dst_hbm, ew_hbm, out_hbm,
              srcv, dstv, ewv, buf0, buf1, gs0, gs1, ss0, ss1,
              acc_sh):
    c = lax.axis_index("c")
    s = lax.axis_index("s")
    bufs = (buf0, buf1)
    gs = (gs0, gs1)
    ss = (ss0, ss1)
    nblk = jnp.where(c == 0, _K0, _K1)
    stripe = acc_sh.shape[0] // _NS
    sizes = [_CW] * (stripe // _CW) + ([stripe % _CW] if stripe % _CW else [])

    def zb(i, carry):
        for q in range(8):
            buf0[i, pl.ds(q * 16, 16)] = jnp.zeros((16,), jnp.float32)
        return carry
    lax.fori_loop(0, _CW, zb, 0)
    off = 0
    for sz in sizes:
        pltpu.sync_copy(buf0.at[pl.ds(0, sz)],
                        acc_sh.at[pl.ds(s * stripe + off, sz)])
        off += sz
    plsc.subcore_barrier()

    base = jnp.where(c == 0, _K0 * s, _K0 * _NS + _K1 * s) * _BLK
    dummy = u_hbm.at[pl.ds(0, _CW)]

    def wait_d(sem, buf):
        pltpu.make_async_copy(dummy, buf, sem).wait()

    def block(b, carry):
        @pl.when(b > 0)
        def _drain():
            for k in range(_NB):
                wait_d(ss[k], bufs[k])
        pltpu.sync_copy(src_hbm.at[pl.ds(base + b * _BLK, _BLK)], srcv)
        pltpu.sync_copy(dst_hbm.at[pl.ds(base + b * _BLK, _BLK)], dstv)
        pltpu.sync_copy(ew_hbm.at[pl.ds(base + b * _BLK, _BLK)], ewv)
        for k in range(_NB):
            pltpu.async_copy(u_hbm.at[srcv.at[k]], bufs[k], gs[k])

        def quad(q, c2):
            for k in range(_NB):
                r = _NB * q + k
                wait_d(gs[k], bufs[k])
                _scale(bufs[k], ewv, r)
                pltpu.async_copy(bufs[k], acc_sh.at[dstv.at[r]], ss[k],
                                 add=True)

                @pl.when(q < _BLK // _NB - 1)
                def _next():
                    wait_d(ss[k], bufs[k])
                    pltpu.async_copy(u_hbm.at[srcv.at[r + _NB]], bufs[k],
                                     gs[k])
            return c2
        lax.fori_loop(0, _BLK // _NB, quad, 0)
        return carry
    lax.fori_loop(0, nblk, block, 0)

    @pl.when(nblk > 0)
    def _final_drain():
        for k in range(_NB):
            wait_d(ss[k], bufs[k])
    plsc.subcore_barrier()

    off = 0
    for sz in sizes:
        pltpu.sync_copy(acc_sh.at[pl.ds(s * stripe + off, sz)],
                        buf0.at[pl.ds(0, sz)])
        pltpu.sync_copy(buf0.at[pl.ds(0, sz)],
                        out_hbm.at[c, pl.ds(s * stripe + off, sz)])
        off += sz


def _msg_call(u, src2d, dst2d, ew2d, n2):
    n, d = u.shape
    mesh = plsc.VectorSubcoreMesh(core_axis_name="c", subcore_axis_name="s")
    kfn = pl.kernel(
        functools.partial(_msg_body),
        mesh=mesh,
        out_type=jax.ShapeDtypeStruct((_NC, n2, d), jnp.float32),
        scratch_types=[
            pltpu.VMEM((_BLK, _CW), jnp.int32),
            pltpu.VMEM((_BLK, _CW), jnp.int32),
            pltpu.VMEM((_BLK, _CW), jnp.float32),
            pltpu.VMEM((_CW, d), jnp.float32),
            pltpu.VMEM((_CW, d), jnp.float32),
            pltpu.SemaphoreType.DMA,
            pltpu.SemaphoreType.DMA,
            pltpu.SemaphoreType.DMA,
            pltpu.SemaphoreType.DMA,
            pltpu.VMEM_SHARED((n2, d), jnp.float32),
        ],
    )
    return kfn(u, src2d, dst2d, ew2d)


def _k2_body(degp_ref, x_ref, w_ref, u_ref, dinv_ref):
    deg = degp_ref[0] + degp_ref[1] + 1.0
    dinv = jnp.where(deg > 0, lax.rsqrt(deg), 0.0)
    dinv_ref[...] = dinv
    u_ref[...] = jnp.dot(x_ref[...], w_ref[...],
                         preferred_element_type=jnp.float32) * dinv[:, None]


def _k4_body(mp_ref, u_ref, dinv_ref, b_ref, w_ref, u2_ref):
    dinv = dinv_ref[...]
    h = (mp_ref[0] + mp_ref[1] + u_ref[...]) * dinv[:, None] + b_ref[...][None, :]
    h = jnp.maximum(h, 0.0)
    u2_ref[...] = jnp.dot(h, w_ref[...],
                          preferred_element_type=jnp.float32) * dinv[:, None]


def _k5_body(mp_ref, u_ref, dinv_ref, b_ref, batch_ref, md_ref,
             wm_ref, bm_ref, wfp_ref, wfm_ref, bf_ref, o_ref):
    n = u_ref.shape[0]
    g = md_ref.shape[0]
    dinv = dinv_ref[...]
    h = (mp_ref[0] + mp_ref[1] + u_ref[...]) * dinv[:, None] + b_ref[...][None, :]
    h = jnp.maximum(h, 0.0)
    bt = batch_ref[...]
    gi = lax.broadcasted_iota(jnp.int32, (g, n), 0)
    onehot = (gi == bt[None, :]).astype(jnp.float32)
    sums = jnp.dot(onehot, h, preferred_element_type=jnp.float32)
    counts = jnp.sum(onehot, axis=1)
    pooled = sums / jnp.maximum(counts, 1.0)[:, None]
    first_node = jnp.sum((bt[None, :] < gi).astype(jnp.int32), axis=1)
    idx = lax.rem(first_node, g)
    oh2 = (lax.broadcasted_iota(jnp.int32, (g, g), 1) == idx[:, None]
           ).astype(jnp.float32)
    md = jnp.dot(oh2, md_ref[...], preferred_element_type=jnp.float32)
    md = jnp.maximum(jnp.dot(md, wm_ref[...],
                             preferred_element_type=jnp.float32)
                     + bm_ref[...][None, :], 0.0)
    o_ref[...] = (jnp.dot(pooled, wfp_ref[...], preferred_element_type=jnp.float32)
                  + jnp.dot(md, wfm_ref[...], preferred_element_type=jnp.float32)
                  + bf_ref[...][None, :])


def kernel(x, edge_index, edge_attr, batch, metadata, W1, b1, W2, b2,
           Wm, bm, Wf, bf):
    n, d = x.shape
    e = edge_attr.shape[0]
    g = metadata.shape[0]
    unit = _NS * (_K0 + _K1) * _BLK * _CW
    ep = -(-e // unit) * unit
    pad = ep - e
    src_p = jnp.concatenate([edge_index[0],
                             jnp.zeros((pad,), edge_index.dtype)])
    dst_p = jnp.concatenate([edge_index[1],
                             jnp.zeros((pad,), edge_index.dtype)])
    ew_p = jnp.concatenate([edge_attr, jnp.zeros((pad,), edge_attr.dtype)])
    src2d = src_p.reshape(-1, _CW)
    dst2d = dst_p.reshape(-1, _CW)
    ew2d = ew_p.reshape(-1, _CW)

    n2 = ((n + _W - 1) // _W) * _W
    degp = _deg_call(dst_p.reshape(-1, _W), ew_p.reshape(-1, _W),
                     n2).reshape(_NC, n2)[:, :n]

    u1, dinv = pl.pallas_call(
        _k2_body,
        out_shape=[jax.ShapeDtypeStruct((n, d), jnp.float32),
                   jax.ShapeDtypeStruct((n,), jnp.float32)],
    )(degp, x, W1)

    m1 = _msg_call(u1, src2d, dst2d, ew2d, n2)[:, :n]

    u2 = pl.pallas_call(
        _k4_body,
        out_shape=jax.ShapeDtypeStruct((n, d), jnp.float32),
    )(m1, u1, dinv, b1, W2)

    m2 = _msg_call(u2, src2d, dst2d, ew2d, n2)[:, :n]

    out = pl.pallas_call(
        _k5_body,
        out_shape=jax.ShapeDtypeStruct((g, Wf.shape[1]), jnp.float32),
    )(m2, u2, dinv, b2, batch, metadata, Wm, bm, Wf[:d], Wf[d:], bf)
    return out


# speedup vs baseline: 3.1978x; 1.9866x over previous
import functools

import jax
import jax.numpy as jnp
from jax import lax
from jax.experimental import pallas as pl
from jax.experimental.pallas import tpu as pltpu
from jax.experimental.pallas import tpu_sc as plsc

_NC = 2
_NS = 16
_W = 128
_CW = 128
_BLK = 16
_NB = 2
_K0 = 6
_K1 = 4


def _deg_body(dst_hbm, ew_hbm, out_hbm, idx_v, ew_v, stage_v, acc_sh):
    c = lax.axis_index("c")
    s = lax.axis_index("s")
    n = stage_v.shape[0]
    rows = idx_v.shape[0]

    @pl.when(s == 0)
    def _zero():
        def zb(i, carry):
            stage_v[pl.ds(i * 16, 16)] = jnp.zeros((16,), jnp.float32)
            return carry
        lax.fori_loop(0, n // 16, zb, 0)
        pltpu.sync_copy(stage_v, acc_sh)

    plsc.subcore_barrier()

    base = (c * _NS + s) * rows
    pltpu.sync_copy(dst_hbm.at[pl.ds(base, rows)], idx_v)
    pltpu.sync_copy(ew_hbm.at[pl.ds(base, rows)], ew_v)

    def body(j, carry):
        pltpu.sync_copy(ew_v.at[j], acc_sh.at[idx_v.at[j]], add=True)
        return carry
    lax.fori_loop(0, rows, body, 0)

    plsc.subcore_barrier()

    @pl.when(s == 0)
    def _writeback():
        pltpu.sync_copy(acc_sh, stage_v)
        pltpu.sync_copy(stage_v, out_hbm.at[pl.ds(c * n, n)])


def _deg_call(dst2d, ew2d, n2):
    rows = dst2d.shape[0] // (_NC * _NS)
    mesh = plsc.VectorSubcoreMesh(core_axis_name="c", subcore_axis_name="s")
    kfn = pl.kernel(
        functools.partial(_deg_body),
        mesh=mesh,
        out_type=jax.ShapeDtypeStruct((_NC * n2,), jnp.float32),
        scratch_types=[
            pltpu.VMEM((rows, _W), jnp.int32),
            pltpu.VMEM((rows, _W), jnp.float32),
            pltpu.VMEM((n2,), jnp.float32),
            pltpu.VMEM_SHARED((n2,), jnp.float32),
        ],
    )
    return kfn(dst2d, ew2d)


def _scale(buf, ewv, r):
    def grp(eb, c2):
        wv = ewv[r, pl.ds(eb * 16, 16)]
        for l in range(16):
            wb = lax.broadcast_in_dim(wv[l], (16,), ())
            e = eb * 16 + l
            for q in range(8):
                buf[e, pl.ds(q * 16, 16)] = buf[e, pl.ds(q * 16, 16)] * wb
        return c2
    lax.fori_loop(0, buf.shape[0] // 16, grp, 0)


def _msg_body(u_hbm, src_hbm, dst_hbm, ew_hbm, out_hbm,
              srcv, dstv, ewv, buf0, buf1, gs0, gs1, ss0, ss1,
              acc_sh):
    c = lax.axis_index("c")
    s = lax.axis_index("s")
    bufs = (buf0, buf1)
    gs = (gs0, gs1)
    ss = (ss0, ss1)
    nblk = jnp.where(c == 0, _K0, _K1)
    stripe = acc_sh.shape[0] // _NS
    sizes = [_CW] * (stripe // _CW) + ([stripe % _CW] if stripe % _CW else [])

    def zb(i, carry):
        for q in range(8):
            buf0[i, pl.ds(q * 16, 16)] = jnp.zeros((16,), jnp.float32)
        return carry
    lax.fori_loop(0, _CW, zb, 0)
    off = 0
    for sz in sizes:
        pltpu.sync_copy(buf0.at[pl.ds(0, sz)],
                        acc_sh.at[pl.ds(s * stripe + off, sz)])
        off += sz
    plsc.subcore_barrier()

    base = jnp.where(c == 0, _K0 * s, _K0 * _NS + _K1 * s) * _BLK
    dummy = u_hbm.at[pl.ds(0, _CW)]

    def wait_d(sem, buf):
        pltpu.make_async_copy(dummy, buf, sem).wait()

    def block(b, carry):
        @pl.when(b > 0)
        def _drain():
            for k in range(_NB):
                wait_d(ss[k], bufs[k])
        pltpu.sync_copy(src_hbm.at[pl.ds(base + b * _BLK, _BLK)], srcv)
        pltpu.sync_copy(dst_hbm.at[pl.ds(base + b * _BLK, _BLK)], dstv)
        pltpu.sync_copy(ew_hbm.at[pl.ds(base + b * _BLK, _BLK)], ewv)
        for k in range(_NB):
            pltpu.async_copy(u_hbm.at[srcv.at[k]], bufs[k], gs[k])

        def quad(q, c2):
            for k in range(_NB):
                r = _NB * q + k
                wait_d(gs[k], bufs[k])
                _scale(bufs[k], ewv, r)
                pltpu.async_copy(bufs[k], acc_sh.at[dstv.at[r]], ss[k],
                                 add=True)

                @pl.when(q < _BLK // _NB - 1)
                def _next():
                    wait_d(ss[k], bufs[k])
                    pltpu.async_copy(u_hbm.at[srcv.at[r + _NB]], bufs[k],
                                     gs[k])
            return c2
        lax.fori_loop(0, _BLK // _NB, quad, 0)
        return carry
    lax.fori_loop(0, nblk, block, 0)

    @pl.when(nblk > 0)
    def _final_drain():
        for k in range(_NB):
            wait_d(ss[k], bufs[k])
    plsc.subcore_barrier()

    off = 0
    for sz in sizes:
        pltpu.sync_copy(acc_sh.at[pl.ds(s * stripe + off, sz)],
                        buf0.at[pl.ds(0, sz)])
        pltpu.sync_copy(buf0.at[pl.ds(0, sz)],
                        out_hbm.at[c, pl.ds(s * stripe + off, sz)])
        off += sz


def _msg_call(u, src2d, dst2d, ew2d, n2):
    n, d = u.shape
    mesh = plsc.VectorSubcoreMesh(core_axis_name="c", subcore_axis_name="s")
    kfn = pl.kernel(
        functools.partial(_msg_body),
        mesh=mesh,
        out_type=jax.ShapeDtypeStruct((_NC, n2, d), jnp.float32),
        scratch_types=[
            pltpu.VMEM((_BLK, _CW), jnp.int32),
            pltpu.VMEM((_BLK, _CW), jnp.int32),
            pltpu.VMEM((_BLK, _CW), jnp.float32),
            pltpu.VMEM((_CW, d), jnp.float32),
            pltpu.VMEM((_CW, d), jnp.float32),
            pltpu.SemaphoreType.DMA,
            pltpu.SemaphoreType.DMA,
            pltpu.SemaphoreType.DMA,
            pltpu.SemaphoreType.DMA,
            pltpu.VMEM_SHARED((n2, d), jnp.float32),
        ],
    )
    return kfn(u, src2d, dst2d, ew2d)


def _k2_body(degp_ref, x_ref, w_ref, u_ref, dinv_ref):
    deg = degp_ref[0] + degp_ref[1] + 1.0
    dinv = jnp.where(deg > 0, lax.rsqrt(deg), 0.0)
    dinv_ref[...] = dinv
    u_ref[...] = jnp.dot(x_ref[...], w_ref[...],
                         preferred_element_type=jnp.float32) * dinv[:, None]


def _k4_body(mp_ref, u_ref, dinv_ref, b_ref, w_ref, u2_ref):
    dinv = dinv_ref[...]
    h = (mp_ref[0] + mp_ref[1] + u_ref[...]) * dinv[:, None] + b_ref[...][None, :]
    h = jnp.maximum(h, 0.0)
    u2_ref[...] = jnp.dot(h, w_ref[...],
                          preferred_element_type=jnp.float32) * dinv[:, None]


def _k5_body(mp_ref, u_ref, dinv_ref, b_ref, batch_ref, md_ref,
             wm_ref, bm_ref, wfp_ref, wfm_ref, bf_ref, o_ref):
    n = u_ref.shape[0]
    g = md_ref.shape[0]
    dinv = dinv_ref[...]
    h = (mp_ref[0] + mp_ref[1] + u_ref[...]) * dinv[:, None] + b_ref[...][None, :]
    h = jnp.maximum(h, 0.0)
    bt = batch_ref[...]
    gi = lax.broadcasted_iota(jnp.int32, (g, n), 0)
    onehot = (gi == bt[None, :]).astype(jnp.float32)
    sums = jnp.dot(onehot, h, preferred_element_type=jnp.float32)
    counts = jnp.sum(onehot, axis=1)
    pooled = sums / jnp.maximum(counts, 1.0)[:, None]
    first_node = jnp.sum((bt[None, :] < gi).astype(jnp.int32), axis=1)
    idx = lax.rem(first_node, g)
    oh2 = (lax.broadcasted_iota(jnp.int32, (g, g), 1) == idx[:, None]
           ).astype(jnp.float32)
    md = jnp.dot(oh2, md_ref[...], preferred_element_type=jnp.float32)
    md = jnp.maximum(jnp.dot(md, wm_ref[...],
                             preferred_element_type=jnp.float32)
                     + bm_ref[...][None, :], 0.0)
    o_ref[...] = (jnp.dot(pooled, wfp_ref[...], preferred_element_type=jnp.float32)
                  + jnp.dot(md, wfm_ref[...], preferred_element_type=jnp.float32)
                  + bf_ref[...][None, :])


def kernel(x, edge_index, edge_attr, batch, metadata, W1, b1, W2, b2,
           Wm, bm, Wf, bf):
    n, d = x.shape
    e = edge_attr.shape[0]
    g = metadata.shape[0]
    unit = _NS * (_K0 + _K1) * _BLK * _CW
    ep = -(-e // unit) * unit
    pad = ep - e
    spread = (jnp.arange(pad, dtype=edge_index.dtype) * 8) % n
    src_p = jnp.concatenate([edge_index[0], spread])
    dst_p = jnp.concatenate([edge_index[1], spread])
    ew_p = jnp.concatenate([edge_attr, jnp.zeros((pad,), edge_attr.dtype)])
    src2d = src_p.reshape(-1, _CW)
    dst2d = dst_p.reshape(-1, _CW)
    ew2d = ew_p.reshape(-1, _CW)

    n2 = ((n + _W - 1) // _W) * _W
    degp = _deg_call(dst_p.reshape(-1, _W), ew_p.reshape(-1, _W),
                     n2).reshape(_NC, n2)[:, :n]

    u1, dinv = pl.pallas_call(
        _k2_body,
        out_shape=[jax.ShapeDtypeStruct((n, d), jnp.float32),
                   jax.ShapeDtypeStruct((n,), jnp.float32)],
    )(degp, x, W1)

    m1 = _msg_call(u1, src2d, dst2d, ew2d, n2)[:, :n]

    u2 = pl.pallas_call(
        _k4_body,
        out_shape=jax.ShapeDtypeStruct((n, d), jnp.float32),
    )(m1, u1, dinv, b1, W2)

    m2 = _msg_call(u2, src2d, dst2d, ew2d, n2)[:, :n]

    out = pl.pallas_call(
        _k5_body,
        out_shape=jax.ShapeDtypeStruct((g, Wf.shape[1]), jnp.float32),
    )(m2, u2, dinv, b2, batch, metadata, Wm, bm, Wf[:d], Wf[d:], bf)
    return out


# speedup vs baseline: 3.6114x; 1.1293x over previous
import functools

import jax
import jax.numpy as jnp
from jax import lax
from jax.experimental import pallas as pl
from jax.experimental.pallas import tpu as pltpu
from jax.experimental.pallas import tpu_sc as plsc

_NC = 2
_NS = 16
_W = 128
_CW = 128
_BLK = 16
_NB = 2
_K0 = 5
_K1 = 5


def _deg_body(dst_hbm, ew_hbm, out_hbm, idx_v, ew_v, stage_v, acc_sh):
    c = lax.axis_index("c")
    s = lax.axis_index("s")
    n = stage_v.shape[0]
    rows = idx_v.shape[0]

    @pl.when(s == 0)
    def _zero():
        def zb(i, carry):
            stage_v[pl.ds(i * 16, 16)] = jnp.zeros((16,), jnp.float32)
            return carry
        lax.fori_loop(0, n // 16, zb, 0)
        pltpu.sync_copy(stage_v, acc_sh)

    plsc.subcore_barrier()

    base = (c * _NS + s) * rows
    pltpu.sync_copy(dst_hbm.at[pl.ds(base, rows)], idx_v)
    pltpu.sync_copy(ew_hbm.at[pl.ds(base, rows)], ew_v)

    def body(j, carry):
        pltpu.sync_copy(ew_v.at[j], acc_sh.at[idx_v.at[j]], add=True)
        return carry
    lax.fori_loop(0, rows, body, 0)

    plsc.subcore_barrier()

    @pl.when(s == 0)
    def _writeback():
        pltpu.sync_copy(acc_sh, stage_v)
        pltpu.sync_copy(stage_v, out_hbm.at[pl.ds(c * n, n)])


def _deg_call(dst2d, ew2d, n2):
    rows = dst2d.shape[0] // (_NC * _NS)
    mesh = plsc.VectorSubcoreMesh(core_axis_name="c", subcore_axis_name="s")
    kfn = pl.kernel(
        functools.partial(_deg_body),
        mesh=mesh,
        out_type=jax.ShapeDtypeStruct((_NC * n2,), jnp.float32),
        scratch_types=[
            pltpu.VMEM((rows, _W), jnp.int32),
            pltpu.VMEM((rows, _W), jnp.float32),
            pltpu.VMEM((n2,), jnp.float32),
            pltpu.VMEM_SHARED((n2,), jnp.float32),
        ],
    )
    return kfn(dst2d, ew2d)


def _scale(buf, ewv, r):
    def grp(eb, c2):
        wv = ewv[r, pl.ds(eb * 16, 16)]
        for l in range(16):
            wb = lax.broadcast_in_dim(wv[l], (16,), ())
            e = eb * 16 + l
            for q in range(8):
                buf[e, pl.ds(q * 16, 16)] = buf[e, pl.ds(q * 16, 16)] * wb
        return c2
    lax.fori_loop(0, buf.shape[0] // 16, grp, 0)


def _msg_body(u_hbm, src_hbm, dst_hbm, ew_hbm, out_hbm,
              srcv, dstv, ewv, buf0, buf1, gs0, gs1, ss0, ss1,
              acc_sh):
    c = lax.axis_index("c")
    s = lax.axis_index("s")
    bufs = (buf0, buf1)
    gs = (gs0, gs1)
    ss = (ss0, ss1)
    nblk = jnp.where(c == 0, _K0, _K1)
    stripe = acc_sh.shape[0] // _NS
    sizes = [_CW] * (stripe // _CW) + ([stripe % _CW] if stripe % _CW else [])

    def zb(i, carry):
        for q in range(8):
            buf0[i, pl.ds(q * 16, 16)] = jnp.zeros((16,), jnp.float32)
        return carry
    lax.fori_loop(0, _CW, zb, 0)
    off = 0
    for sz in sizes:
        pltpu.sync_copy(buf0.at[pl.ds(0, sz)],
                        acc_sh.at[pl.ds(s * stripe + off, sz)])
        off += sz
    plsc.subcore_barrier()

    base = jnp.where(c == 0, _K0 * s, _K0 * _NS + _K1 * s) * _BLK
    dummy = u_hbm.at[pl.ds(0, _CW)]

    def wait_d(sem, buf):
        pltpu.make_async_copy(dummy, buf, sem).wait()

    def block(b, carry):
        @pl.when(b > 0)
        def _drain():
            for k in range(_NB):
                wait_d(ss[k], bufs[k])
        pltpu.sync_copy(src_hbm.at[pl.ds(base + b * _BLK, _BLK)], srcv)
        pltpu.sync_copy(dst_hbm.at[pl.ds(base + b * _BLK, _BLK)], dstv)
        pltpu.sync_copy(ew_hbm.at[pl.ds(base + b * _BLK, _BLK)], ewv)
        for k in range(_NB):
            pltpu.async_copy(u_hbm.at[srcv.at[k]], bufs[k], gs[k])

        def quad(q, c2):
            for k in range(_NB):
                r = _NB * q + k
                wait_d(gs[k], bufs[k])
                _scale(bufs[k], ewv, r)
                pltpu.async_copy(bufs[k], acc_sh.at[dstv.at[r]], ss[k],
                                 add=True)

                @pl.when(q < _BLK // _NB - 1)
                def _next():
                    wait_d(ss[k], bufs[k])
                    pltpu.async_copy(u_hbm.at[srcv.at[r + _NB]], bufs[k],
                                     gs[k])
            return c2
        lax.fori_loop(0, _BLK // _NB, quad, 0)
        return carry
    lax.fori_loop(0, nblk, block, 0)

    @pl.when(nblk > 0)
    def _final_drain():
        for k in range(_NB):
            wait_d(ss[k], bufs[k])
    plsc.subcore_barrier()

    off = 0
    for sz in sizes:
        pltpu.sync_copy(acc_sh.at[pl.ds(s * stripe + off, sz)],
                        buf0.at[pl.ds(0, sz)])
        pltpu.sync_copy(buf0.at[pl.ds(0, sz)],
                        out_hbm.at[c, pl.ds(s * stripe + off, sz)])
        off += sz


def _msg_call(u, src2d, dst2d, ew2d, n2):
    n, d = u.shape
    mesh = plsc.VectorSubcoreMesh(core_axis_name="c", subcore_axis_name="s")
    kfn = pl.kernel(
        functools.partial(_msg_body),
        mesh=mesh,
        out_type=jax.ShapeDtypeStruct((_NC, n2, d), jnp.float32),
        scratch_types=[
            pltpu.VMEM((_BLK, _CW), jnp.int32),
            pltpu.VMEM((_BLK, _CW), jnp.int32),
            pltpu.VMEM((_BLK, _CW), jnp.float32),
            pltpu.VMEM((_CW, d), jnp.float32),
            pltpu.VMEM((_CW, d), jnp.float32),
            pltpu.SemaphoreType.DMA,
            pltpu.SemaphoreType.DMA,
            pltpu.SemaphoreType.DMA,
            pltpu.SemaphoreType.DMA,
            pltpu.VMEM_SHARED((n2, d), jnp.float32),
        ],
    )
    return kfn(u, src2d, dst2d, ew2d)


def _k2_body(degp_ref, x_ref, w_ref, u_ref, dinv_ref):
    deg = degp_ref[0] + degp_ref[1] + 1.0
    dinv = jnp.where(deg > 0, lax.rsqrt(deg), 0.0)
    dinv_ref[...] = dinv
    u_ref[...] = jnp.dot(x_ref[...], w_ref[...],
                         preferred_element_type=jnp.float32) * dinv[:, None]


def _k4_body(mp_ref, u_ref, dinv_ref, b_ref, w_ref, u2_ref):
    dinv = dinv_ref[...]
    h = (mp_ref[0] + mp_ref[1] + u_ref[...]) * dinv[:, None] + b_ref[...][None, :]
    h = jnp.maximum(h, 0.0)
    u2_ref[...] = jnp.dot(h, w_ref[...],
                          preferred_element_type=jnp.float32) * dinv[:, None]


def _k5_body(mp_ref, u_ref, dinv_ref, b_ref, batch_ref, md_ref,
             wm_ref, bm_ref, wfp_ref, wfm_ref, bf_ref, o_ref):
    n = u_ref.shape[0]
    g = md_ref.shape[0]
    dinv = dinv_ref[...]
    h = (mp_ref[0] + mp_ref[1] + u_ref[...]) * dinv[:, None] + b_ref[...][None, :]
    h = jnp.maximum(h, 0.0)
    bt = batch_ref[...]
    gi = lax.broadcasted_iota(jnp.int32, (g, n), 0)
    onehot = (gi == bt[None, :]).astype(jnp.float32)
    sums = jnp.dot(onehot, h, preferred_element_type=jnp.float32)
    counts = jnp.sum(onehot, axis=1)
    pooled = sums / jnp.maximum(counts, 1.0)[:, None]
    first_node = jnp.sum((bt[None, :] < gi).astype(jnp.int32), axis=1)
    idx = lax.rem(first_node, g)
    oh2 = (lax.broadcasted_iota(jnp.int32, (g, g), 1) == idx[:, None]
           ).astype(jnp.float32)
    md = jnp.dot(oh2, md_ref[...], preferred_element_type=jnp.float32)
    md = jnp.maximum(jnp.dot(md, wm_ref[...],
                             preferred_element_type=jnp.float32)
                     + bm_ref[...][None, :], 0.0)
    o_ref[...] = (jnp.dot(pooled, wfp_ref[...], preferred_element_type=jnp.float32)
                  + jnp.dot(md, wfm_ref[...], preferred_element_type=jnp.float32)
                  + bf_ref[...][None, :])


def kernel(x, edge_index, edge_attr, batch, metadata, W1, b1, W2, b2,
           Wm, bm, Wf, bf):
    n, d = x.shape
    e = edge_attr.shape[0]
    g = metadata.shape[0]
    unit = _NS * (_K0 + _K1) * _BLK * _CW
    ep = -(-e // unit) * unit
    pad = ep - e
    spread = (jnp.arange(pad, dtype=edge_index.dtype) * 8) % n
    src_p = jnp.concatenate([edge_index[0], spread])
    dst_p = jnp.concatenate([edge_index[1], spread])
    ew_p = jnp.concatenate([edge_attr, jnp.zeros((pad,), edge_attr.dtype)])
    src2d = src_p.reshape(-1, _CW)
    dst2d = dst_p.reshape(-1, _CW)
    ew2d = ew_p.reshape(-1, _CW)

    n2 = ((n + _W - 1) // _W) * _W
    degp = _deg_call(dst_p.reshape(-1, _W), ew_p.reshape(-1, _W),
                     n2).reshape(_NC, n2)[:, :n]

    u1, dinv = pl.pallas_call(
        _k2_body,
        out_shape=[jax.ShapeDtypeStruct((n, d), jnp.float32),
                   jax.ShapeDtypeStruct((n,), jnp.float32)],
    )(degp, x, W1)

    m1 = _msg_call(u1, src2d, dst2d, ew2d, n2)[:, :n]

    u2 = pl.pallas_call(
        _k4_body,
        out_shape=jax.ShapeDtypeStruct((n, d), jnp.float32),
    )(m1, u1, dinv, b1, W2)

    m2 = _msg_call(u2, src2d, dst2d, ew2d, n2)[:, :n]

    out = pl.pallas_call(
        _k5_body,
        out_shape=jax.ShapeDtypeStruct((g, Wf.shape[1]), jnp.float32),
    )(m2, u2, dinv, b2, batch, metadata, Wm, bm, Wf[:d], Wf[d:], bf)
    return out


# speedup vs baseline: 3.7379x; 1.0350x over previous
import functools

import jax
import jax.numpy as jnp
from jax import lax
from jax.experimental import pallas as pl
from jax.experimental.pallas import tpu as pltpu
from jax.experimental.pallas import tpu_sc as plsc

_NC = 2
_NS = 16
_W = 128
_CW = 128
_BLK = 16
_NB = 2
_K0 = 5
_K1 = 5


def _deg_body(dst_hbm, ew_hbm, out_hbm, idx_v, ew_v, stage_v, acc_sh):
    c = lax.axis_index("c")
    s = lax.axis_index("s")
    n = stage_v.shape[0]
    rows = idx_v.shape[0]

    @pl.when(s == 0)
    def _zero():
        def zb(i, carry):
            stage_v[pl.ds(i * 16, 16)] = jnp.zeros((16,), jnp.float32)
            return carry
        lax.fori_loop(0, n // 16, zb, 0)
        pltpu.sync_copy(stage_v, acc_sh)

    plsc.subcore_barrier()

    base = (c * _NS + s) * rows
    pltpu.sync_copy(dst_hbm.at[pl.ds(base, rows)], idx_v)
    pltpu.sync_copy(ew_hbm.at[pl.ds(base, rows)], ew_v)

    def body(j, carry):
        pltpu.sync_copy(ew_v.at[j], acc_sh.at[idx_v.at[j]], add=True)
        return carry
    lax.fori_loop(0, rows, body, 0)

    plsc.subcore_barrier()

    @pl.when(s == 0)
    def _writeback():
        pltpu.sync_copy(acc_sh, stage_v)
        pltpu.sync_copy(stage_v, out_hbm.at[pl.ds(c * n, n)])


def _deg_call(dst2d, ew2d, n2):
    rows = dst2d.shape[0] // (_NC * _NS)
    mesh = plsc.VectorSubcoreMesh(core_axis_name="c", subcore_axis_name="s")
    kfn = pl.kernel(
        functools.partial(_deg_body),
        mesh=mesh,
        out_type=jax.ShapeDtypeStruct((_NC * n2,), jnp.float32),
        scratch_types=[
            pltpu.VMEM((rows, _W), jnp.int32),
            pltpu.VMEM((rows, _W), jnp.float32),
            pltpu.VMEM((n2,), jnp.float32),
            pltpu.VMEM_SHARED((n2,), jnp.float32),
        ],
    )
    return kfn(dst2d, ew2d)


def _scale(buf, ewv, r):
    def grp(eb, c2):
        wv = ewv[r, pl.ds(eb * 16, 16)]
        for l in range(16):
            wb = lax.broadcast_in_dim(wv[l], (16,), ())
            e = eb * 16 + l
            for q in range(8):
                buf[e, pl.ds(q * 16, 16)] = buf[e, pl.ds(q * 16, 16)] * wb
        return c2
    lax.fori_loop(0, buf.shape[0] // 16, grp, 0)


def _msg_body(u_hbm, src_hbm, dst_hbm, ew_hbm, out_hbm,
              srcv, dstv, ewv, buf0, buf1, gs0, gs1, ss0, ss1,
              acc_sh):
    c = lax.axis_index("c")
    s = lax.axis_index("s")
    bufs = (buf0, buf1)
    gs = (gs0, gs1)
    ss = (ss0, ss1)
    nblk = jnp.where(c == 0, _K0, _K1)
    stripe = acc_sh.shape[0] // _NS
    sizes = [_CW] * (stripe // _CW) + ([stripe % _CW] if stripe % _CW else [])

    def zb(i, carry):
        for q in range(8):
            buf0[i, pl.ds(q * 16, 16)] = jnp.zeros((16,), jnp.float32)
        return carry
    lax.fori_loop(0, _CW, zb, 0)
    off = 0
    for sz in sizes:
        pltpu.sync_copy(buf0.at[pl.ds(0, sz)],
                        acc_sh.at[pl.ds(s * stripe + off, sz)])
        off += sz
    plsc.subcore_barrier()

    base = jnp.where(c == 0, _K0 * s, _K0 * _NS + _K1 * s) * _BLK
    dummy = u_hbm.at[pl.ds(0, _CW)]

    def wait_d(sem, buf):
        pltpu.make_async_copy(dummy, buf, sem).wait()

    def block(b, carry):
        @pl.when(b > 0)
        def _drain():
            for k in range(_NB):
                wait_d(ss[k], bufs[k])
        pltpu.sync_copy(src_hbm.at[pl.ds(base + b * _BLK, _BLK)], srcv)
        pltpu.sync_copy(dst_hbm.at[pl.ds(base + b * _BLK, _BLK)], dstv)
        pltpu.sync_copy(ew_hbm.at[pl.ds(base + b * _BLK, _BLK)], ewv)
        for k in range(_NB):
            pltpu.async_copy(u_hbm.at[srcv.at[k]], bufs[k], gs[k])

        def quad(q, c2):
            for k in range(_NB):
                r = _NB * q + k
                wait_d(gs[k], bufs[k])
                _scale(bufs[k], ewv, r)
                pltpu.async_copy(bufs[k], acc_sh.at[dstv.at[r]], ss[k],
                                 add=True)

                @pl.when(q < _BLK // _NB - 1)
                def _next():
                    wait_d(ss[k], bufs[k])
                    pltpu.async_copy(u_hbm.at[srcv.at[r + _NB]], bufs[k],
                                     gs[k])
            return c2
        lax.fori_loop(0, _BLK // _NB, quad, 0)
        return carry
    lax.fori_loop(0, nblk, block, 0)

    @pl.when(nblk > 0)
    def _final_drain():
        for k in range(_NB):
            wait_d(ss[k], bufs[k])
    plsc.subcore_barrier()

    off = 0
    for sz in sizes:
        pltpu.sync_copy(acc_sh.at[pl.ds(s * stripe + off, sz)],
                        buf0.at[pl.ds(0, sz)])
        pltpu.sync_copy(buf0.at[pl.ds(0, sz)],
                        out_hbm.at[c, pl.ds(s * stripe + off, sz)])
        off += sz


def _msg_call(u, src2d, dst2d, ew2d, n2):
    n, d = u.shape
    mesh = plsc.VectorSubcoreMesh(core_axis_name="c", subcore_axis_name="s")
    kfn = pl.kernel(
        functools.partial(_msg_body),
        mesh=mesh,
        out_type=jax.ShapeDtypeStruct((_NC, n2, d), jnp.float32),
        scratch_types=[
            pltpu.VMEM((_BLK, _CW), jnp.int32),
            pltpu.VMEM((_BLK, _CW), jnp.int32),
            pltpu.VMEM((_BLK, _CW), jnp.float32),
            pltpu.VMEM((_CW, d), jnp.float32),
            pltpu.VMEM((_CW, d), jnp.float32),
            pltpu.SemaphoreType.DMA,
            pltpu.SemaphoreType.DMA,
            pltpu.SemaphoreType.DMA,
            pltpu.SemaphoreType.DMA,
            pltpu.VMEM_SHARED((n2, d), jnp.float32),
        ],
    )
    return kfn(u, src2d, dst2d, ew2d)


def _k2a_body(x_ref, w_ref, t_ref):
    t_ref[...] = jnp.dot(x_ref[...], w_ref[...],
                         preferred_element_type=jnp.float32)


def _k2b_body(degp_ref, t_ref, u_ref, dinv_ref):
    n = t_ref.shape[0]
    deg = degp_ref[0, :n] + degp_ref[1, :n] + 1.0
    dinv = jnp.where(deg > 0, lax.rsqrt(deg), 0.0)
    dinv_ref[...] = dinv
    u_ref[...] = t_ref[...] * dinv[:, None]


def _k4_body(mp_ref, u_ref, dinv_ref, b_ref, w_ref, u2_ref):
    n = u_ref.shape[0]
    dinv = dinv_ref[...]
    h = (mp_ref[0, :n] + mp_ref[1, :n] + u_ref[...]) * dinv[:, None] + b_ref[...][None, :]
    h = jnp.maximum(h, 0.0)
    u2_ref[...] = jnp.dot(h, w_ref[...],
                          preferred_element_type=jnp.float32) * dinv[:, None]


def _k5_body(mp_ref, u_ref, dinv_ref, b_ref, batch_ref, md_ref,
             wm_ref, bm_ref, wfp_ref, wfm_ref, bf_ref, o_ref):
    n = u_ref.shape[0]
    g = md_ref.shape[0]
    dinv = dinv_ref[...]
    h = (mp_ref[0, :n] + mp_ref[1, :n] + u_ref[...]) * dinv[:, None] + b_ref[...][None, :]
    h = jnp.maximum(h, 0.0)
    bt = batch_ref[...]
    gi = lax.broadcasted_iota(jnp.int32, (g, n), 0)
    onehot = (gi == bt[None, :]).astype(jnp.float32)
    sums = jnp.dot(onehot, h, preferred_element_type=jnp.float32)
    counts = jnp.sum(onehot, axis=1)
    pooled = sums / jnp.maximum(counts, 1.0)[:, None]
    first_node = jnp.sum((bt[None, :] < gi).astype(jnp.int32), axis=1)
    idx = lax.rem(first_node, g)
    oh2 = (lax.broadcasted_iota(jnp.int32, (g, g), 1) == idx[:, None]
           ).astype(jnp.float32)
    md = jnp.dot(oh2, md_ref[...], preferred_element_type=jnp.float32)
    md = jnp.maximum(jnp.dot(md, wm_ref[...],
                             preferred_element_type=jnp.float32)
                     + bm_ref[...][None, :], 0.0)
    o_ref[...] = (jnp.dot(pooled, wfp_ref[...], preferred_element_type=jnp.float32)
                  + jnp.dot(md, wfm_ref[...], preferred_element_type=jnp.float32)
                  + bf_ref[...][None, :])


def kernel(x, edge_index, edge_attr, batch, metadata, W1, b1, W2, b2,
           Wm, bm, Wf, bf):
    n, d = x.shape
    e = edge_attr.shape[0]
    g = metadata.shape[0]
    unit = _NS * (_K0 + _K1) * _BLK * _CW
    ep = -(-e // unit) * unit
    pad = ep - e
    spread = (jnp.arange(pad, dtype=edge_index.dtype) * 8) % n
    src_p = jnp.concatenate([edge_index[0], spread])
    dst_p = jnp.concatenate([edge_index[1], spread])
    ew_p = jnp.concatenate([edge_attr, jnp.zeros((pad,), edge_attr.dtype)])
    src2d = src_p.reshape(-1, _CW)
    dst2d = dst_p.reshape(-1, _CW)
    ew2d = ew_p.reshape(-1, _CW)

    n2 = ((n + _W - 1) // _W) * _W
    degp = _deg_call(dst_p.reshape(-1, _W), ew_p.reshape(-1, _W),
                     n2).reshape(_NC, n2)

    t1 = pl.pallas_call(
        _k2a_body,
        out_shape=jax.ShapeDtypeStruct((n, d), jnp.float32),
    )(x, W1)
    u1, dinv = pl.pallas_call(
        _k2b_body,
        out_shape=[jax.ShapeDtypeStruct((n, d), jnp.float32),
                   jax.ShapeDtypeStruct((n,), jnp.float32)],
    )(degp, t1)

    m1 = _msg_call(u1, src2d, dst2d, ew2d, n2)

    u2 = pl.pallas_call(
        _k4_body,
        out_shape=jax.ShapeDtypeStruct((n, d), jnp.float32),
    )(m1, u1, dinv, b1, W2)

    m2 = _msg_call(u2, src2d, dst2d, ew2d, n2)

    out = pl.pallas_call(
        _k5_body,
        out_shape=jax.ShapeDtypeStruct((g, Wf.shape[1]), jnp.float32),
    )(m2, u2, dinv, b2, batch, metadata, Wm, bm, Wf[:d], Wf[d:], bf)
    return out
